# baseline (speedup 1.0000x reference)
import functools

import jax
import jax.numpy as jnp
from jax import lax
from jax.experimental import pallas as pl
from jax.experimental.pallas import tpu as pltpu
from jax.experimental.pallas import tpu_sc as plsc

N_NODES_C = 66
N_EDGES_C = 2048
BATCH_C = 3
PER_BATCH_C = 22
NPAD_C = 128

_SC_CORES = 1
_SC_SUBCORES = 16
_NW = _SC_CORES * _SC_SUBCORES
_EPW = N_EDGES_C // _NW

_HI = lax.Precision.HIGHEST


_ZCH = (128 * 128) // 16


def _sc_adj_kernel(src_hbm, dst_hbm, out_hbm, ssta, dsta, eidx, ones, zbuf, acc):
    cid = lax.axis_index("c")
    sid = lax.axis_index("s")
    wid = sid * _SC_CORES + cid
    base = wid * _EPW

    pltpu.sync_copy(src_hbm.at[pl.ds(base, _EPW)], ssta)
    pltpu.sync_copy(dst_hbm.at[pl.ds(base, _EPW)], dsta)

    ones16 = jnp.ones((16,), jnp.float32)
    for j in range(_EPW // 16):
        sl = pl.ds(j * 16, 16)
        eidx[sl] = dsta[sl] * NPAD_C + ssta[sl]
        ones[sl] = ones16

    zeros16 = jnp.zeros((16,), jnp.float32)

    def _z(i, carry):
        zbuf[pl.ds(i * 16, 16)] = zeros16
        return carry

    lax.fori_loop(0, _ZCH // 16, _z, 0)
    pltpu.sync_copy(zbuf, acc.at[pl.ds(sid * _ZCH, _ZCH)])

    plsc.subcore_barrier()

    pltpu.sync_copy(ones, acc.at[eidx], add=True)
    plsc.subcore_barrier()

    pltpu.sync_copy(acc.at[pl.ds(sid * _ZCH, _ZCH)],
                    out_hbm.at[cid, pl.ds(sid * _ZCH, _ZCH)])


_sc_adj = functools.partial(
    pl.kernel,
    out_type=jax.ShapeDtypeStruct((_SC_CORES, NPAD_C * NPAD_C), jnp.float32),
    mesh=plsc.VectorSubcoreMesh(core_axis_name="c", subcore_axis_name="s",
                                num_cores=_SC_CORES),
    scratch_types=[
        pltpu.VMEM((_EPW,), jnp.int32),
        pltpu.VMEM((_EPW,), jnp.int32),
        pltpu.VMEM((_EPW,), jnp.int32),
        pltpu.VMEM((_EPW,), jnp.float32),
        pltpu.VMEM((_ZCH,), jnp.float32),
        pltpu.VMEM_SHARED((NPAD_C * NPAD_C,), jnp.float32),
    ],
)(_sc_adj_kernel)


def _gcn_kernel(a_ref, f_ref, wl_ref, bl_ref,
                w1_ref, b1_ref, w2_ref, b2_ref, w3_ref, b3_ref,
                wr_ref, bro_ref, out_ref):
    A = a_ref[0, :, :]

    f_pad = jnp.concatenate(
        [f_ref[:, :], jnp.zeros((NPAD_C - N_NODES_C, f_ref.shape[1]),
                                jnp.float32)], axis=0)
    x = lax.dot_general(f_pad, wl_ref[:, :], (((1,), (1,)), ((), ())),
                        precision=_HI, preferred_element_type=jnp.float32)
    x = jnp.maximum(x + bl_ref[:, :], 0.0)

    for w_ref, b_ref in ((w1_ref, b1_ref), (w2_ref, b2_ref), (w3_ref, b3_ref)):
        agg = lax.dot_general(A, x, (((1,), (0,)), ((), ())),
                              precision=_HI, preferred_element_type=jnp.float32)
        x = lax.dot_general(agg, w_ref[:, :], (((1,), (1,)), ((), ())),
                            precision=_HI, preferred_element_type=jnp.float32)
        x = jnp.maximum(x + b_ref[:, :], 0.0)

    wro = wr_ref[:, :]
    dh = x.shape[1]
    wrows = [wro[:, j * dh:(j + 1) * dh] for j in range(PER_BATCH_C)]
    wr_tiled = jnp.concatenate(
        wrows * BATCH_C
        + [jnp.zeros((NPAD_C - N_NODES_C, dh), jnp.float32)], axis=0)
    weighted = x * wr_tiled
    rows = jnp.sum(weighted, axis=1, keepdims=True)
    sel_n = lax.broadcasted_iota(jnp.int32, (BATCH_C, NPAD_C), 1)
    sel_b = lax.broadcasted_iota(jnp.int32, (BATCH_C, NPAD_C), 0)
    S = (sel_n // PER_BATCH_C == sel_b).astype(jnp.float32)
    out = lax.dot_general(S, rows, (((1,), (0,)), ((), ())),
                          precision=_HI, preferred_element_type=jnp.float32)
    out_ref[:, :] = out + bro_ref[:, :]


def kernel(features, edge_index, W_lift, b_lift, W1, b1, W2, b2, W3, b3, W_ro, b_ro):
    a_halves = _sc_adj(edge_index[0], edge_index[1])
    a_halves = a_halves.reshape(_SC_CORES, NPAD_C, NPAD_C)
    out = pl.pallas_call(
        _gcn_kernel,
        out_shape=jax.ShapeDtypeStruct((BATCH_C, 1), jnp.float32),
    )(a_halves, features, W_lift, b_lift.reshape(1, -1),
      W1, b1.reshape(1, -1), W2, b2.reshape(1, -1), W3, b3.reshape(1, -1),
      W_ro, b_ro.reshape(1, 1))
    return out


# speedup vs baseline: 1.0005x; 1.0005x over previous
import functools

import jax
import jax.numpy as jnp
from jax import lax
from jax.experimental import pallas as pl
from jax.experimental.pallas import tpu as pltpu
from jax.experimental.pallas import tpu_sc as plsc

N_NODES_C = 66
N_EDGES_C = 2048
BATCH_C = 3
PER_BATCH_C = 22
NPAD_C = 128

_SC_CORES = 1
_SC_SUBCORES = 16
_NW = _SC_CORES * _SC_SUBCORES
_EPW = N_EDGES_C // _NW

_HI = lax.Precision.HIGHEST


_ZCH = (128 * 128) // 16


def _sc_adj_kernel(edge_hbm, out_hbm, ssta, dsta, eidx, ones, zbuf, acc):
    cid = lax.axis_index("c")
    sid = lax.axis_index("s")
    wid = sid * _SC_CORES + cid
    base = wid * _EPW

    pltpu.sync_copy(edge_hbm.at[0, pl.ds(base, _EPW)], ssta)
    pltpu.sync_copy(edge_hbm.at[1, pl.ds(base, _EPW)], dsta)

    ones16 = jnp.ones((16,), jnp.float32)
    for j in range(_EPW // 16):
        sl = pl.ds(j * 16, 16)
        eidx[sl] = dsta[sl] * NPAD_C + ssta[sl]
        ones[sl] = ones16

    zeros16 = jnp.zeros((16,), jnp.float32)

    def _z(i, carry):
        zbuf[pl.ds(i * 16, 16)] = zeros16
        return carry

    lax.fori_loop(0, _ZCH // 16, _z, 0)
    pltpu.sync_copy(zbuf, acc.at[pl.ds(sid * _ZCH, _ZCH)])

    plsc.subcore_barrier()

    pltpu.sync_copy(ones, acc.at[eidx], add=True)
    plsc.subcore_barrier()

    pltpu.sync_copy(acc.at[pl.ds(sid * _ZCH, _ZCH)],
                    out_hbm.at[cid, pl.ds(sid * _ZCH, _ZCH)])


_sc_adj = functools.partial(
    pl.kernel,
    out_type=jax.ShapeDtypeStruct((_SC_CORES, NPAD_C * NPAD_C), jnp.float32),
    mesh=plsc.VectorSubcoreMesh(core_axis_name="c", subcore_axis_name="s",
                                num_cores=_SC_CORES),
    scratch_types=[
        pltpu.VMEM((_EPW,), jnp.int32),
        pltpu.VMEM((_EPW,), jnp.int32),
        pltpu.VMEM((_EPW,), jnp.int32),
        pltpu.VMEM((_EPW,), jnp.float32),
        pltpu.VMEM((_ZCH,), jnp.float32),
        pltpu.VMEM_SHARED((NPAD_C * NPAD_C,), jnp.float32),
    ],
)(_sc_adj_kernel)


def _gcn_kernel(a_ref, f_ref, wl_ref, bl_ref,
                w1_ref, b1_ref, w2_ref, b2_ref, w3_ref, b3_ref,
                wr_ref, bro_ref, out_ref):
    A = a_ref[0, :, :]

    f_pad = jnp.concatenate(
        [f_ref[:, :], jnp.zeros((NPAD_C - N_NODES_C, f_ref.shape[1]),
                                jnp.float32)], axis=0)
    x = lax.dot_general(f_pad, wl_ref[:, :], (((1,), (1,)), ((), ())),
                        precision=_HI, preferred_element_type=jnp.float32)
    x = jnp.maximum(x + bl_ref[:, :], 0.0)

    for w_ref, b_ref in ((w1_ref, b1_ref), (w2_ref, b2_ref), (w3_ref, b3_ref)):
        agg = lax.dot_general(A, x, (((1,), (0,)), ((), ())),
                              precision=_HI, preferred_element_type=jnp.float32)
        x = lax.dot_general(agg, w_ref[:, :], (((1,), (1,)), ((), ())),
                            precision=_HI, preferred_element_type=jnp.float32)
        x = jnp.maximum(x + b_ref[:, :], 0.0)

    wro = wr_ref[:, :]
    dh = x.shape[1]
    wrows = [wro[:, j * dh:(j + 1) * dh] for j in range(PER_BATCH_C)]
    wr_tiled = jnp.concatenate(
        wrows * BATCH_C
        + [jnp.zeros((NPAD_C - N_NODES_C, dh), jnp.float32)], axis=0)
    weighted = x * wr_tiled
    rows = jnp.sum(weighted, axis=1, keepdims=True)
    sel_n = lax.broadcasted_iota(jnp.int32, (BATCH_C, NPAD_C), 1)
    sel_b = lax.broadcasted_iota(jnp.int32, (BATCH_C, NPAD_C), 0)
    S = (sel_n // PER_BATCH_C == sel_b).astype(jnp.float32)
    out = lax.dot_general(S, rows, (((1,), (0,)), ((), ())),
                          precision=_HI, preferred_element_type=jnp.float32)
    out_ref[:, :] = out + bro_ref[:, :]


def kernel(features, edge_index, W_lift, b_lift, W1, b1, W2, b2, W3, b3, W_ro, b_ro):
    a_halves = _sc_adj(edge_index)
    a_halves = a_halves.reshape(_SC_CORES, NPAD_C, NPAD_C)
    out = pl.pallas_call(
        _gcn_kernel,
        out_shape=jax.ShapeDtypeStruct((BATCH_C, 1), jnp.float32),
    )(a_halves, features, W_lift, b_lift.reshape(1, -1),
      W1, b1.reshape(1, -1), W2, b2.reshape(1, -1), W3, b3.reshape(1, -1),
      W_ro, b_ro.reshape(1, 1))
    return out


# speedup vs baseline: 1.0536x; 1.0531x over previous
import functools

import jax
import jax.numpy as jnp
from jax import lax
from jax.experimental import pallas as pl
from jax.experimental.pallas import tpu as pltpu
from jax.experimental.pallas import tpu_sc as plsc

N_NODES_C = 66
N_EDGES_C = 2048
BATCH_C = 3
PER_BATCH_C = 22
NPAD_C = 128

_SC_CORES = 1
_SC_SUBCORES = 16
_NW = _SC_CORES * _SC_SUBCORES
_EPW = N_EDGES_C // _NW

_HI = lax.Precision.HIGHEST


_ZCH = (128 * 128) // 16


def _sc_adj_kernel(edge_hbm, out_hbm, ssta, dsta, eidx, ones, zbuf, acc):
    cid = lax.axis_index("c")
    sid = lax.axis_index("s")
    wid = sid * _SC_CORES + cid
    base = wid * _EPW

    pltpu.sync_copy(edge_hbm.at[0, pl.ds(base, _EPW)], ssta)
    pltpu.sync_copy(edge_hbm.at[1, pl.ds(base, _EPW)], dsta)

    ones16 = jnp.ones((16,), jnp.float32)
    for j in range(_EPW // 16):
        sl = pl.ds(j * 16, 16)
        eidx[sl] = dsta[sl] * NPAD_C + ssta[sl]
        ones[sl] = ones16

    zeros16 = jnp.zeros((16,), jnp.float32)

    def _z(i, carry):
        zbuf[pl.ds(i * 16, 16)] = zeros16
        return carry

    lax.fori_loop(0, _ZCH // 16, _z, 0)
    pltpu.sync_copy(zbuf, acc.at[pl.ds(sid * _ZCH, _ZCH)])

    plsc.subcore_barrier()

    pltpu.sync_copy(ones, acc.at[eidx], add=True)
    plsc.subcore_barrier()

    pltpu.sync_copy(acc.at[pl.ds(sid * _ZCH, _ZCH)],
                    out_hbm.at[cid, pl.ds(sid * _ZCH, _ZCH)])


_sc_adj = functools.partial(
    pl.kernel,
    out_type=jax.ShapeDtypeStruct((_SC_CORES, NPAD_C * NPAD_C), jnp.float32),
    mesh=plsc.VectorSubcoreMesh(core_axis_name="c", subcore_axis_name="s",
                                num_cores=_SC_CORES),
    scratch_types=[
        pltpu.VMEM((_EPW,), jnp.int32),
        pltpu.VMEM((_EPW,), jnp.int32),
        pltpu.VMEM((_EPW,), jnp.int32),
        pltpu.VMEM((_EPW,), jnp.float32),
        pltpu.VMEM((_ZCH,), jnp.float32),
        pltpu.VMEM_SHARED((NPAD_C * NPAD_C,), jnp.float32),
    ],
)(_sc_adj_kernel)


def _gcn_kernel(a_ref, f_ref, wl_ref, bl_ref,
                w1_ref, b1_ref, w2_ref, b2_ref, w3_ref, b3_ref,
                wr_ref, bro_ref, out_ref):
    A = a_ref[0, :, :]

    f_pad = jnp.concatenate(
        [f_ref[:, :], jnp.zeros((NPAD_C - N_NODES_C, f_ref.shape[1]),
                                jnp.float32)], axis=0)
    x = lax.dot_general(f_pad, wl_ref[:, :], (((1,), (1,)), ((), ())),
                        precision=_HI, preferred_element_type=jnp.float32)
    x = jnp.maximum(x + bl_ref[:, :], 0.0)

    for w_ref, b_ref in ((w1_ref, b1_ref), (w2_ref, b2_ref), (w3_ref, b3_ref)):
        agg = lax.dot_general(A, x, (((1,), (0,)), ((), ())),
                              preferred_element_type=jnp.float32)
        x = lax.dot_general(agg, w_ref[:, :], (((1,), (1,)), ((), ())),
                            preferred_element_type=jnp.float32)
        x = jnp.maximum(x + b_ref[:, :], 0.0)

    wro = wr_ref[:, :]
    dh = x.shape[1]
    wrows = [wro[:, j * dh:(j + 1) * dh] for j in range(PER_BATCH_C)]
    wr_tiled = jnp.concatenate(
        wrows * BATCH_C
        + [jnp.zeros((NPAD_C - N_NODES_C, dh), jnp.float32)], axis=0)
    weighted = x * wr_tiled
    rows = jnp.sum(weighted, axis=1, keepdims=True)
    sel_n = lax.broadcasted_iota(jnp.int32, (BATCH_C, NPAD_C), 1)
    sel_b = lax.broadcasted_iota(jnp.int32, (BATCH_C, NPAD_C), 0)
    S = (sel_n // PER_BATCH_C == sel_b).astype(jnp.float32)
    out = lax.dot_general(S, rows, (((1,), (0,)), ((), ())),
                          precision=_HI, preferred_element_type=jnp.float32)
    out_ref[:, :] = out + bro_ref[:, :]


def kernel(features, edge_index, W_lift, b_lift, W1, b1, W2, b2, W3, b3, W_ro, b_ro):
    a_halves = _sc_adj(edge_index)
    a_halves = a_halves.reshape(_SC_CORES, NPAD_C, NPAD_C)
    out = pl.pallas_call(
        _gcn_kernel,
        out_shape=jax.ShapeDtypeStruct((BATCH_C, 1), jnp.float32),
    )(a_halves, features, W_lift, b_lift.reshape(1, -1),
      W1, b1.reshape(1, -1), W2, b2.reshape(1, -1), W3, b3.reshape(1, -1),
      W_ro, b_ro.reshape(1, 1))
    return out
